# baseline (speedup 1.0000x reference)
import jax
import jax.numpy as jnp
from jax.experimental import pallas as pl
from jax.experimental.pallas import tpu as pltpu

N = 5000
BLK = 512
PAD = 5120
NPROG = PAD // BLK
ENC = 8192.0
PERM = (9, 0, 1, 8, 7, 2, 3, 6, 5, 4)


def _nms_rank_kernel(perm_ref, packed, row_ref, col_ref):
    blk = perm_ref[pl.program_id(0)]
    i0 = blk * BLK

    blkT = jnp.transpose(packed[:, pl.ds(i0, BLK)], (1, 0))
    rx1 = blkT[:, 0:1]
    ry1 = blkT[:, 1:2]
    rx2 = blkT[:, 2:3]
    ry2 = blkT[:, 3:4]
    rs = blkT[:, 4:5]
    rarea = (rx2 - rx1) * (ry2 - ry1)

    C = BLK

    def chunk(c0):
        cx1 = packed[0:1, pl.ds(c0, C)]
        cy1 = packed[1:2, pl.ds(c0, C)]
        cx2 = packed[2:3, pl.ds(c0, C)]
        cy2 = packed[3:4, pl.ds(c0, C)]
        cs = packed[4:5, pl.ds(c0, C)]
        iw = jnp.maximum(jnp.minimum(rx2, cx2) - jnp.maximum(rx1, cx1), 0.0)
        ih = jnp.maximum(jnp.minimum(ry2, cy2) - jnp.maximum(ry1, cy1), 0.0)
        inter = iw * ih
        carea = (cx2 - cx1) * (cy2 - cy1)
        union = (rarea + carea) - inter
        m = (inter + inter) - union
        t = jnp.where(m > 0.0, ENC + 1.0, 1.0)
        cge = cs >= rs
        return t, cge

    def saturate(s):
        vf = jnp.floor(s * (1.0 / ENC))
        return (s - vf * ENC) + ENC * jnp.minimum(vf, 1.0)

    col_ref[0, :, :] = jnp.zeros((1, PAD), jnp.float32)

    def body(c, acc):
        c0 = c * C
        t, cge = chunk(c0)
        cr = jnp.where(cge, t, 0.0)
        acc = acc + saturate(jnp.sum(cr, axis=1, keepdims=True))
        col_ref[0, :, pl.ds(c0, C)] = saturate(
            jnp.sum(t - cr, axis=0, keepdims=True)
        )
        return acc

    acc = jnp.zeros((BLK, 1), dtype=jnp.float32)
    acc = jax.lax.fori_loop(0, blk, body, acc)

    tri = (
        jax.lax.broadcasted_iota(jnp.int32, (1, C), 1)
        < jax.lax.broadcasted_iota(jnp.int32, (BLK, 1), 0)
    )
    t, cge = chunk(i0)
    cr = jnp.where(jnp.logical_and(tri, cge), t, 0.0)
    acc = acc + saturate(jnp.sum(cr, axis=1, keepdims=True))
    col_ref[0, :, pl.ds(i0, C)] = saturate(
        jnp.sum(jnp.where(tri, t, 0.0) - cr, axis=0, keepdims=True)
    )

    row_ref[:, :] = jnp.transpose(acc, (1, 0))


def kernel(boxes, scores):
    pad = PAD - N
    packed = jnp.concatenate(
        [boxes.T, scores[None, :], jnp.zeros((3, N), jnp.float32)], axis=0
    )
    packed = jnp.pad(packed, ((0, 0), (0, pad)))
    perm = jnp.array(PERM, dtype=jnp.int32)

    grid_spec = pltpu.PrefetchScalarGridSpec(
        num_scalar_prefetch=1,
        grid=(NPROG,),
        in_specs=[pl.BlockSpec((8, PAD), lambda i, p: (0, 0))],
        out_specs=[
            pl.BlockSpec((1, BLK), lambda i, p: (0, p[i])),
            pl.BlockSpec((1, 1, PAD), lambda i, p: (i, 0, 0)),
        ],
    )
    row_enc, colpart = pl.pallas_call(
        _nms_rank_kernel,
        grid_spec=grid_spec,
        out_shape=[
            jax.ShapeDtypeStruct((1, PAD), jnp.float32),
            jax.ShapeDtypeStruct((NPROG, 1, PAD), jnp.float32),
        ],
        compiler_params=pltpu.CompilerParams(
            dimension_semantics=("parallel",)
        ),
    )(perm, packed)

    enc = row_enc[0, :] + jnp.sum(colpart[:, 0, :], axis=0)
    nviol = jnp.floor(enc * (1.0 / ENC))
    rank = (enc - nviol * ENC).astype(jnp.int32)
    keep = jnp.where(nviol > 0.0, 0.0, 1.0)
    valsT = packed[:5] * keep[None, :]
    outT = jnp.zeros((5, PAD), jnp.float32).at[:, rank].set(valsT, unique_indices=True)
    return outT[:, :N].T


# speedup vs baseline: 1.0221x; 1.0221x over previous
import jax
import jax.numpy as jnp
from jax.experimental import pallas as pl

N = 5000
BLK = 512
PAD = 5120
ENC = 8192.0


def _nms_rank_kernel(packed, row_ref, col_ref):
    i = pl.program_id(0)
    i0 = i * BLK

    blkT = jnp.transpose(packed[:, pl.ds(i0, BLK)], (1, 0))
    rx1 = blkT[:, 0:1]
    ry1 = blkT[:, 1:2]
    rx2 = blkT[:, 2:3]
    ry2 = blkT[:, 3:4]
    rs = blkT[:, 4:5]
    rarea = (rx2 - rx1) * (ry2 - ry1)

    C = BLK

    def chunk(c0):
        cx1 = packed[0:1, pl.ds(c0, C)]
        cy1 = packed[1:2, pl.ds(c0, C)]
        cx2 = packed[2:3, pl.ds(c0, C)]
        cy2 = packed[3:4, pl.ds(c0, C)]
        cs = packed[4:5, pl.ds(c0, C)]
        iw = jnp.maximum(jnp.minimum(rx2, cx2) - jnp.maximum(rx1, cx1), 0.0)
        ih = jnp.maximum(jnp.minimum(ry2, cy2) - jnp.maximum(ry1, cy1), 0.0)
        inter = iw * ih
        carea = (cx2 - cx1) * (cy2 - cy1)
        union = (rarea + carea) - inter
        m = (inter + inter) - union
        t = jnp.where(m > 0.0, ENC + 1.0, 1.0)
        cge = cs >= rs
        return t, cge

    def saturate(s):
        vf = jnp.floor(s * (1.0 / ENC))
        return (s - vf * ENC) + ENC * jnp.minimum(vf, 1.0)

    def body(c, acc):
        c0 = c * C
        t, cge = chunk(c0)
        cr = jnp.where(cge, t, 0.0)
        acc = acc + saturate(jnp.sum(cr, axis=1, keepdims=True))
        col_ref[:, pl.ds(c0, C)] = col_ref[:, pl.ds(c0, C)] + saturate(
            jnp.sum(t - cr, axis=0, keepdims=True)
        )
        return acc

    acc = jnp.zeros((BLK, 1), dtype=jnp.float32)
    acc = jax.lax.fori_loop(0, i, body, acc)

    tri = (
        jax.lax.broadcasted_iota(jnp.int32, (1, C), 1)
        < jax.lax.broadcasted_iota(jnp.int32, (BLK, 1), 0)
    )
    t, cge = chunk(i0)
    cr = jnp.where(jnp.logical_and(tri, cge), t, 0.0)
    acc = acc + saturate(jnp.sum(cr, axis=1, keepdims=True))
    col_ref[:, pl.ds(i0, C)] = saturate(
        jnp.sum(jnp.where(tri, t, 0.0) - cr, axis=0, keepdims=True)
    )

    row_ref[:, :] = jnp.transpose(acc, (1, 0))


def kernel(boxes, scores):
    pad = PAD - N
    packed = jnp.concatenate(
        [boxes.T, scores[None, :], jnp.zeros((3, N), jnp.float32)], axis=0
    )
    packed = jnp.pad(packed, ((0, 0), (0, pad)))

    row_enc, col_enc = pl.pallas_call(
        _nms_rank_kernel,
        grid=(PAD // BLK,),
        in_specs=[pl.BlockSpec((8, PAD), lambda i: (0, 0))],
        out_specs=[
            pl.BlockSpec((1, BLK), lambda i: (0, i)),
            pl.BlockSpec((1, PAD), lambda i: (0, 0)),
        ],
        out_shape=[
            jax.ShapeDtypeStruct((1, PAD), jnp.float32),
            jax.ShapeDtypeStruct((1, PAD), jnp.float32),
        ],
    )(packed)

    enc = row_enc[0, :] + col_enc[0, :]
    nviol = jnp.floor(enc * (1.0 / ENC))
    rank = (enc - nviol * ENC).astype(jnp.int32)
    keep = jnp.where(nviol > 0.0, 0.0, 1.0)
    valsT = packed[:5] * keep[None, :]
    outT = jnp.zeros((5, PAD), jnp.float32).at[:, rank].set(valsT, unique_indices=True)
    return outT[:, :N].T


# speedup vs baseline: 1.1298x; 1.1054x over previous
import jax
import jax.numpy as jnp
from jax.experimental import pallas as pl

N = 5000
BLK = 1024
PAD = 5120
ENC = 8192.0


def _nms_rank_kernel(packed, row_ref, col_ref):
    i = pl.program_id(0)
    i0 = i * BLK

    blkT = jnp.transpose(packed[:, pl.ds(i0, BLK)], (1, 0))
    rx1 = blkT[:, 0:1]
    ry1 = blkT[:, 1:2]
    rx2 = blkT[:, 2:3]
    ry2 = blkT[:, 3:4]
    rs = blkT[:, 4:5]
    rarea = (rx2 - rx1) * (ry2 - ry1)

    C = BLK

    def chunk(c0):
        cx1 = packed[0:1, pl.ds(c0, C)]
        cy1 = packed[1:2, pl.ds(c0, C)]
        cx2 = packed[2:3, pl.ds(c0, C)]
        cy2 = packed[3:4, pl.ds(c0, C)]
        cs = packed[4:5, pl.ds(c0, C)]
        iw = jnp.maximum(jnp.minimum(rx2, cx2) - jnp.maximum(rx1, cx1), 0.0)
        ih = jnp.maximum(jnp.minimum(ry2, cy2) - jnp.maximum(ry1, cy1), 0.0)
        inter = iw * ih
        carea = (cx2 - cx1) * (cy2 - cy1)
        union = (rarea + carea) - inter
        m = (inter + inter) - union
        t = jnp.where(m > 0.0, ENC + 1.0, 1.0)
        cge = cs >= rs
        return t, cge

    def saturate(s):
        vf = jnp.floor(s * (1.0 / ENC))
        return (s - vf * ENC) + ENC * jnp.minimum(vf, 1.0)

    def body(c, acc):
        c0 = c * C
        t, cge = chunk(c0)
        cr = jnp.where(cge, t, 0.0)
        acc = acc + saturate(jnp.sum(cr, axis=1, keepdims=True))
        col_ref[:, pl.ds(c0, C)] = col_ref[:, pl.ds(c0, C)] + saturate(
            jnp.sum(t - cr, axis=0, keepdims=True)
        )
        return acc

    acc = jnp.zeros((BLK, 1), dtype=jnp.float32)
    acc = jax.lax.fori_loop(0, i, body, acc)

    tri = (
        jax.lax.broadcasted_iota(jnp.int32, (1, C), 1)
        < jax.lax.broadcasted_iota(jnp.int32, (BLK, 1), 0)
    )
    t, cge = chunk(i0)
    cr = jnp.where(jnp.logical_and(tri, cge), t, 0.0)
    acc = acc + saturate(jnp.sum(cr, axis=1, keepdims=True))
    col_ref[:, pl.ds(i0, C)] = saturate(
        jnp.sum(jnp.where(tri, t, 0.0) - cr, axis=0, keepdims=True)
    )

    row_ref[:, :] = jnp.transpose(acc, (1, 0))


def kernel(boxes, scores):
    pad = PAD - N
    packed = jnp.concatenate(
        [boxes.T, scores[None, :], jnp.zeros((3, N), jnp.float32)], axis=0
    )
    packed = jnp.pad(packed, ((0, 0), (0, pad)))

    row_enc, col_enc = pl.pallas_call(
        _nms_rank_kernel,
        grid=(PAD // BLK,),
        in_specs=[pl.BlockSpec((8, PAD), lambda i: (0, 0))],
        out_specs=[
            pl.BlockSpec((1, BLK), lambda i: (0, i)),
            pl.BlockSpec((1, PAD), lambda i: (0, 0)),
        ],
        out_shape=[
            jax.ShapeDtypeStruct((1, PAD), jnp.float32),
            jax.ShapeDtypeStruct((1, PAD), jnp.float32),
        ],
    )(packed)

    enc = row_enc[0, :] + col_enc[0, :]
    nviol = jnp.floor(enc * (1.0 / ENC))
    rank = (enc - nviol * ENC).astype(jnp.int32)
    keep = jnp.where(nviol > 0.0, 0.0, 1.0)
    valsT = packed[:5] * keep[None, :]
    outT = jnp.zeros((5, PAD), jnp.float32).at[:, rank].set(valsT, unique_indices=True)
    return outT[:, :N].T


# speedup vs baseline: 1.1596x; 1.0264x over previous
import jax
import jax.numpy as jnp
from jax.experimental import pallas as pl

N = 5000
BLK = 1280
PAD = 5120
ENC = 8192.0


def _nms_rank_kernel(packed, row_ref, col_ref):
    i = pl.program_id(0)
    i0 = i * BLK

    blkT = jnp.transpose(packed[:, pl.ds(i0, BLK)], (1, 0))
    rx1 = blkT[:, 0:1]
    ry1 = blkT[:, 1:2]
    rx2 = blkT[:, 2:3]
    ry2 = blkT[:, 3:4]
    rs = blkT[:, 4:5]
    rarea = (rx2 - rx1) * (ry2 - ry1)

    C = BLK

    def chunk(c0):
        cx1 = packed[0:1, pl.ds(c0, C)]
        cy1 = packed[1:2, pl.ds(c0, C)]
        cx2 = packed[2:3, pl.ds(c0, C)]
        cy2 = packed[3:4, pl.ds(c0, C)]
        cs = packed[4:5, pl.ds(c0, C)]
        iw = jnp.maximum(jnp.minimum(rx2, cx2) - jnp.maximum(rx1, cx1), 0.0)
        ih = jnp.maximum(jnp.minimum(ry2, cy2) - jnp.maximum(ry1, cy1), 0.0)
        inter = iw * ih
        carea = (cx2 - cx1) * (cy2 - cy1)
        union = (rarea + carea) - inter
        m = (inter + inter) - union
        t = jnp.where(m > 0.0, ENC + 1.0, 1.0)
        cge = cs >= rs
        return t, cge

    def saturate(s):
        vf = jnp.floor(s * (1.0 / ENC))
        return (s - vf * ENC) + ENC * jnp.minimum(vf, 1.0)

    def body(c, acc):
        c0 = c * C
        t, cge = chunk(c0)
        cr = jnp.where(cge, t, 0.0)
        acc = acc + saturate(jnp.sum(cr, axis=1, keepdims=True))
        col_ref[:, pl.ds(c0, C)] = col_ref[:, pl.ds(c0, C)] + saturate(
            jnp.sum(t - cr, axis=0, keepdims=True)
        )
        return acc

    acc = jnp.zeros((BLK, 1), dtype=jnp.float32)
    acc = jax.lax.fori_loop(0, i, body, acc)

    tri = (
        jax.lax.broadcasted_iota(jnp.int32, (1, C), 1)
        < jax.lax.broadcasted_iota(jnp.int32, (BLK, 1), 0)
    )
    t, cge = chunk(i0)
    cr = jnp.where(jnp.logical_and(tri, cge), t, 0.0)
    acc = acc + saturate(jnp.sum(cr, axis=1, keepdims=True))
    col_ref[:, pl.ds(i0, C)] = saturate(
        jnp.sum(jnp.where(tri, t, 0.0) - cr, axis=0, keepdims=True)
    )

    row_ref[:, :] = jnp.transpose(acc, (1, 0))


def kernel(boxes, scores):
    pad = PAD - N
    packed = jnp.concatenate(
        [boxes.T, scores[None, :], jnp.zeros((3, N), jnp.float32)], axis=0
    )
    packed = jnp.pad(packed, ((0, 0), (0, pad)))

    row_enc, col_enc = pl.pallas_call(
        _nms_rank_kernel,
        grid=(PAD // BLK,),
        in_specs=[pl.BlockSpec((8, PAD), lambda i: (0, 0))],
        out_specs=[
            pl.BlockSpec((1, BLK), lambda i: (0, i)),
            pl.BlockSpec((1, PAD), lambda i: (0, 0)),
        ],
        out_shape=[
            jax.ShapeDtypeStruct((1, PAD), jnp.float32),
            jax.ShapeDtypeStruct((1, PAD), jnp.float32),
        ],
    )(packed)

    enc = row_enc[0, :] + col_enc[0, :]
    nviol = jnp.floor(enc * (1.0 / ENC))
    rank = (enc - nviol * ENC).astype(jnp.int32)
    keep = jnp.where(nviol > 0.0, 0.0, 1.0)
    valsT = packed[:5] * keep[None, :]
    outT = jnp.zeros((5, PAD), jnp.float32).at[:, rank].set(valsT, unique_indices=True)
    return outT[:, :N].T
